# baseline (speedup 1.0000x reference)
import numpy as np
import jax
import jax.numpy as jnp
from jax import lax
from jax.experimental import pallas as pl
from jax.experimental.pallas import tpu as pltpu, tpu_sc as plsc

_JNT = np.array([0, 5, 1, 9, 13, 17, 6, 2, 10, 14, 18, 7, 3, 11, 15, 19, 8, 4, 12, 16, 20])
_PERM = (_JNT[:, None] + np.arange(3)[None, :]).flatten()

_ROWS = 16384
_COLS = 63
_NC = 2
_NS = 16
_NW = _NC * _NS
_RPW = _ROWS // _NW
_PERIOD = 1008
_PROWS = _PERIOD // _COLS
_NBLK = _RPW // _PROWS
_NVEC = _PERIOD // 16
_UNROLL = 4

_PP = np.arange(_PERIOD)
_T_ROW = (_PP // _COLS).astype(np.int32)
_T_OC = (_PP % _COLS).astype(np.int32)
_T_IC = _PERM[_PP % _COLS].astype(np.int32)


def _body(in_hbm, tr_hbm, tic_hbm, toc_hbm, out_hbm, in_v, out_v, tr_v, tic_v, toc_v):
    wid = lax.axis_index("s") * _NC + lax.axis_index("c")
    r0 = wid * _RPW
    pltpu.sync_copy(tr_hbm, tr_v)
    pltpu.sync_copy(tic_hbm, tic_v)
    pltpu.sync_copy(toc_hbm, toc_v)
    pltpu.sync_copy(in_hbm.at[pl.ds(r0, _RPW), :], in_v)

    for v in range(_NVEC):
        o = v * 16
        row0 = tr_v[pl.ds(o, 16)]
        ic = tic_v[pl.ds(o, 16)]
        oc = toc_v[pl.ds(o, 16)]

        def stepk(g, row, ic=ic, oc=oc):
            for u in range(_UNROLL):
                val = plsc.load_gather(in_v, [row + u * _PROWS, ic])
                plsc.store_scatter(out_v, [row + u * _PROWS, oc], val)
            return row + _UNROLL * _PROWS

        lax.fori_loop(0, _NBLK // _UNROLL, stepk, row0)

    pltpu.sync_copy(out_v, out_hbm.at[pl.ds(r0, _RPW), :])


def kernel(inputs):
    mesh = plsc.VectorSubcoreMesh(core_axis_name="c", subcore_axis_name="s")
    out = pl.kernel(
        _body,
        mesh=mesh,
        out_type=jax.ShapeDtypeStruct((_ROWS, _COLS), jnp.float32),
        scratch_types=[
            pltpu.VMEM((_RPW, _COLS), jnp.float32),
            pltpu.VMEM((_RPW, _COLS), jnp.float32),
            pltpu.VMEM((_PERIOD,), jnp.int32),
            pltpu.VMEM((_PERIOD,), jnp.int32),
            pltpu.VMEM((_PERIOD,), jnp.int32),
        ],
        compiler_params=pltpu.CompilerParams(
            needs_layout_passes=False, use_tc_tiling_on_sc=False
        ),
    )(inputs, jnp.asarray(_T_ROW), jnp.asarray(_T_IC), jnp.asarray(_T_OC))
    return (out, jnp.zeros((_ROWS, 1), inputs.dtype))


# speedup vs baseline: 2.1725x; 2.1725x over previous
import numpy as np
import jax
import jax.numpy as jnp
from jax.experimental import pallas as pl
from jax.experimental.pallas import tpu as pltpu

_JNT = np.array([0, 5, 1, 9, 13, 17, 6, 2, 10, 14, 18, 7, 3, 11, 15, 19, 8, 4, 12, 16, 20])
_PERM = (_JNT[:, None] + np.arange(3)[None, :]).flatten()

_ROWS = 16384
_COLS = 63
_BR = 1024
_GRID = _ROWS // _BR

_PSEL = np.zeros((_COLS, _COLS), np.float32)
_PSEL[_PERM, np.arange(_COLS)] = 1.0


def _body(x_ref, p_ref, o_ref, z_ref):
    o_ref[...] = jnp.dot(x_ref[...], p_ref[...], preferred_element_type=jnp.float32)
    z_ref[...] = jnp.zeros_like(z_ref)


def kernel(inputs):
    out, z = pl.pallas_call(
        _body,
        grid=(_GRID,),
        in_specs=[
            pl.BlockSpec((_BR, _COLS), lambda i: (i, 0)),
            pl.BlockSpec((_COLS, _COLS), lambda i: (0, 0)),
        ],
        out_specs=[
            pl.BlockSpec((_BR, _COLS), lambda i: (i, 0)),
            pl.BlockSpec((_BR, 1), lambda i: (i, 0)),
        ],
        out_shape=[
            jax.ShapeDtypeStruct((_ROWS, _COLS), jnp.float32),
            jax.ShapeDtypeStruct((_ROWS, 1), jnp.float32),
        ],
        compiler_params=pltpu.CompilerParams(
            dimension_semantics=("arbitrary",),
        ),
    )(inputs, jnp.asarray(_PSEL))
    return (out, z)


# speedup vs baseline: 6.8767x; 3.1654x over previous
import numpy as np
import jax
import jax.numpy as jnp
from jax.experimental import pallas as pl
from jax.experimental.pallas import tpu as pltpu

_JNT = np.array([0, 5, 1, 9, 13, 17, 6, 2, 10, 14, 18, 7, 3, 11, 15, 19, 8, 4, 12, 16, 20])
_PERM = (_JNT[:, None] + np.arange(3)[None, :]).flatten()

_ROWS = 16384
_COLS = 63
_KSRC = 24
_BC = 1024
_GRID = _ROWS // _BC

_PSEL = np.zeros((_COLS, _KSRC), np.float32)
_PSEL[np.arange(_COLS), _PERM] = 1.0


def _body(p_ref, x_ref, o_ref, z_ref):
    o_ref[...] = jnp.dot(p_ref[...], x_ref[...], preferred_element_type=jnp.float32)
    z_ref[...] = jnp.zeros_like(z_ref)


def kernel(inputs):
    x_t = inputs.T
    out_t, z_t = pl.pallas_call(
        _body,
        grid=(_GRID,),
        in_specs=[
            pl.BlockSpec((_COLS, _KSRC), lambda i: (0, 0)),
            pl.BlockSpec((_KSRC, _BC), lambda i: (0, i)),
        ],
        out_specs=[
            pl.BlockSpec((_COLS, _BC), lambda i: (0, i)),
            pl.BlockSpec((1, _BC), lambda i: (0, i)),
        ],
        out_shape=[
            jax.ShapeDtypeStruct((_COLS, _ROWS), jnp.float32),
            jax.ShapeDtypeStruct((1, _ROWS), jnp.float32),
        ],
        compiler_params=pltpu.CompilerParams(
            dimension_semantics=("arbitrary",),
        ),
    )(jnp.asarray(_PSEL), x_t)
    return (out_t.T, z_t.T)


# speedup vs baseline: 14.8904x; 2.1653x over previous
import numpy as np
import jax
import jax.numpy as jnp
from jax.experimental import pallas as pl
from jax.experimental.pallas import tpu as pltpu

_JNT = np.array([0, 5, 1, 9, 13, 17, 6, 2, 10, 14, 18, 7, 3, 11, 15, 19, 8, 4, 12, 16, 20])
_PERM = (_JNT[:, None] + np.arange(3)[None, :]).flatten()

_ROWS = 16384
_COLS = 63
_KSRC = 24
_BC = 4096
_GRID = _ROWS // _BC

_PSEL = np.zeros((_COLS, _KSRC), np.float32)
_PSEL[np.arange(_COLS), _PERM] = 1.0


def _body(p_ref, x_ref, o_ref, z_ref):
    o_ref[...] = jnp.dot(p_ref[...], x_ref[...], preferred_element_type=jnp.float32)
    z_ref[...] = jnp.zeros_like(z_ref)


def kernel(inputs):
    x_t = inputs.T
    out_t, z_t = pl.pallas_call(
        _body,
        grid=(_GRID,),
        in_specs=[
            pl.BlockSpec((_COLS, _KSRC), lambda i: (0, 0)),
            pl.BlockSpec((_KSRC, _BC), lambda i: (0, i)),
        ],
        out_specs=[
            pl.BlockSpec((_COLS, _BC), lambda i: (0, i)),
            pl.BlockSpec((1, _BC), lambda i: (0, i)),
        ],
        out_shape=[
            jax.ShapeDtypeStruct((_COLS, _ROWS), jnp.float32),
            jax.ShapeDtypeStruct((1, _ROWS), jnp.float32),
        ],
        compiler_params=pltpu.CompilerParams(
            dimension_semantics=("parallel",),
        ),
    )(jnp.asarray(_PSEL), x_t)
    return (out_t.T, z_t.T)
